# Initial kernel scaffold; baseline (speedup 1.0000x reference)
import jax
import jax.numpy as jnp
from jax import lax
from jax.experimental import pallas as pl
from jax.experimental.pallas import tpu as pltpu
from jax.experimental.pallas import tpu_sc as plsc

N = 10000
F = 128
E = 320000
NPAD = 10240
EP = E + N
CH16 = 20736
EP_PAD = 16 * CH16
CH32 = CH16 // 2
NCHUNK = CH32 // 128
NSLICE = NPAD // 16


def _dense_kernel(x_ref, w_ref, as_ref, ad_ref, xp_ref, asrc_ref, adst_ref,
                  c_ref):
    xp = jnp.dot(x_ref[...], w_ref[...], preferred_element_type=jnp.float32)
    xp_ref[...] = xp
    a_s = jnp.sum(xp * as_ref[...], axis=1)
    a_d = jnp.sum(xp * ad_ref[...], axis=1)
    asrc_ref[...] = a_s
    adst_ref[...] = a_d
    c = jnp.max(a_s) + jnp.max(a_d)
    c = jnp.where(c >= 0, c, 0.2 * c)
    c_ref[...] = jnp.full((128,), c, jnp.float32)


def _combine_kernel(p_ref, b_ref, w_ref, o_ref):
    o = p_ref[0] + p_ref[1] + b_ref[...][None, :]
    o_ref[...] = jnp.where(o >= 0, o, w_ref[0, 0] * o)


def _sc_edge_kernel(src_hbm, dst_hbm, asrc_hbm, adst_hbm, c_hbm, xp_hbm,
                    out_hbm,
                    sv, dv, ex, av_s, av_d, dn, rows, ssti, dsti, cbuf, sem,
                    acc, dstage, dfinal):
    cid = lax.axis_index("c")
    sid = lax.axis_index("s")
    zero16 = jnp.zeros((16,), jnp.float32)

    off16 = sid * CH16
    pltpu.sync_copy(src_hbm.at[pl.ds(off16, CH16)], sv)
    pltpu.sync_copy(dst_hbm.at[pl.ds(off16, CH16)], dv)
    pltpu.sync_copy(asrc_hbm, av_s)
    pltpu.sync_copy(adst_hbm, av_d)
    pltpu.sync_copy(c_hbm, cbuf)
    c = cbuf[0]

    def zero_dn(i, carry):
        dn[pl.ds(i * 16, 16)] = zero16
        return carry
    lax.fori_loop(0, NPAD // 16, zero_dn, 0)

    def zero_rows(i, carry):
        r = i // 8
        q = lax.rem(i, 8)
        rows[r, pl.ds(q * 16, 16)] = zero16
        return carry
    lax.fori_loop(0, 128 * 8, zero_rows, 0)
    for k in range(NSLICE // 128):
        pltpu.sync_copy(rows, acc.at[pl.ds(sid * NSLICE + k * 128, 128)])

    def edge_body(i, carry):
        s16 = sv[pl.ds(i * 16, 16)]
        d16 = dv[pl.ds(i * 16, 16)]
        a = plsc.load_gather(av_s, [s16]) + plsc.load_gather(av_d, [d16])
        a = jnp.where(a >= 0, a, 0.2 * a) - c
        e16 = jnp.exp(a)
        ex[pl.ds(i * 16, 16)] = e16
        plsc.addupdate_scatter(dn, [d16], e16)
        return carry
    lax.fori_loop(0, CH16 // 16, edge_body, 0)

    pltpu.sync_copy(dn, dstage.at[sid])
    plsc.subcore_barrier()
    for t in range(16):
        pltpu.sync_copy(dstage.at[t, pl.ds(sid * NSLICE, NSLICE)],
                        av_s.at[pl.ds(t * NSLICE, NSLICE)])

    def sum_body(j, carry):
        tot = zero16
        for t in range(16):
            tot = tot + av_s[pl.ds(t * NSLICE + j * 16, 16)]
        dn[pl.ds(sid * NSLICE + j * 16, 16)] = tot
        return carry
    lax.fori_loop(0, NSLICE // 16, sum_body, 0)
    pltpu.sync_copy(dn.at[pl.ds(sid * NSLICE, NSLICE)],
                    dfinal.at[pl.ds(sid * NSLICE, NSLICE)])
    plsc.subcore_barrier()
    pltpu.sync_copy(dfinal, dn)

    coff = cid * CH32

    def coef_body(k, carry):
        idx = coff + k * 16
        d16 = dv[pl.ds(idx, 16)]
        den = plsc.load_gather(dn, [d16])
        ex[pl.ds(idx, 16)] = ex[pl.ds(idx, 16)] / (den + 1e-16)
        return carry
    lax.fori_loop(0, CH32 // 16, coef_body, 0)

    def chunk_body(jc, carry):
        off = coff + jc * 128
        pltpu.sync_copy(sv.at[pl.ds(off, 128)], ssti)
        pltpu.sync_copy(dv.at[pl.ds(off, 128)], dsti)
        pltpu.async_copy(xp_hbm.at[ssti], rows, sem).wait()

        def scale_body(e, c2):
            base = off + e * 4
            for u in range(4):
                cf = ex[base + u]
                cv = jnp.full((16,), cf, jnp.float32)
                for j in range(8):
                    sl = pl.ds(j * 16, 16)
                    rows[e * 4 + u, sl] = rows[e * 4 + u, sl] * cv
            return c2
        lax.fori_loop(0, 32, scale_body, 0)
        pltpu.sync_copy(rows, acc.at[dsti], add=True)
        return carry
    lax.fori_loop(0, NCHUNK, chunk_body, 0)

    plsc.subcore_barrier()
    pltpu.sync_copy(acc.at[pl.ds(sid * NSLICE, NSLICE)],
                    out_hbm.at[cid, pl.ds(sid * NSLICE, NSLICE)])


def kernel(x, edge_index, W, att_src, att_dst, bias, prelu_w):
    xpad = jnp.zeros((NPAD, F), jnp.float32).at[:N].set(x)
    loop = jnp.arange(N, dtype=jnp.int32)
    pad = jnp.full((EP_PAD - EP,), N, dtype=jnp.int32)
    src = jnp.concatenate([edge_index[0], loop, pad])
    dst = jnp.concatenate([edge_index[1], loop, pad])

    xp, a_src, a_dst, c = pl.pallas_call(
        _dense_kernel,
        out_shape=(
            jax.ShapeDtypeStruct((NPAD, F), jnp.float32),
            jax.ShapeDtypeStruct((NPAD,), jnp.float32),
            jax.ShapeDtypeStruct((NPAD,), jnp.float32),
            jax.ShapeDtypeStruct((128,), jnp.float32),
        ),
    )(xpad, W, att_src.reshape(1, F), att_dst.reshape(1, F))

    mesh = plsc.VectorSubcoreMesh(core_axis_name="c", subcore_axis_name="s")
    sc = pl.kernel(
        _sc_edge_kernel,
        out_type=jax.ShapeDtypeStruct((2, NPAD, F), jnp.float32),
        mesh=mesh,
        scratch_types=[
            pltpu.VMEM((CH16,), jnp.int32),
            pltpu.VMEM((CH16,), jnp.int32),
            pltpu.VMEM((CH16,), jnp.float32),
            pltpu.VMEM((NPAD,), jnp.float32),
            pltpu.VMEM((NPAD,), jnp.float32),
            pltpu.VMEM((NPAD,), jnp.float32),
            pltpu.VMEM((128, F), jnp.float32),
            pltpu.VMEM((128,), jnp.int32),
            pltpu.VMEM((128,), jnp.int32),
            pltpu.VMEM((128,), jnp.float32),
            pltpu.SemaphoreType.DMA,
            pltpu.VMEM_SHARED((NPAD, F), jnp.float32),
            pltpu.VMEM_SHARED((16, NPAD), jnp.float32),
            pltpu.VMEM_SHARED((NPAD,), jnp.float32),
        ],
    )
    partial = sc(src, dst, a_src, a_dst, c, xp)

    out = pl.pallas_call(
        _combine_kernel,
        out_shape=jax.ShapeDtypeStruct((NPAD, F), jnp.float32),
    )(partial, bias, prelu_w.reshape(1, 1))
    return out[:N]


# speedup vs baseline: 15.3859x; 15.3859x over previous
import jax
import jax.numpy as jnp
from jax import lax
from jax.experimental import pallas as pl
from jax.experimental.pallas import tpu as pltpu
from jax.experimental.pallas import tpu_sc as plsc

N = 10000
F = 128
FH = F // 2
E = 320000
NPAD = 10240
EP = E + N
CH16 = 21504
EP_PAD = 16 * CH16
NCHUNK = CH16 // 128
NSLICE = NPAD // 16
DROWS = NPAD // 128


def _dense_kernel(x_ref, w_ref, as_ref, ad_ref, xp_ref, asrc_ref, adst_ref,
                  c_ref):
    xp = jnp.dot(x_ref[...], w_ref[...], preferred_element_type=jnp.float32)
    xp_ref[:NPAD, :] = xp[:, :FH]
    xp_ref[NPAD:, :] = xp[:, FH:]
    a_s = jnp.sum(xp * as_ref[...], axis=1)
    a_d = jnp.sum(xp * ad_ref[...], axis=1)
    asrc_ref[...] = a_s
    adst_ref[...] = a_d
    c = jnp.max(a_s) + jnp.max(a_d)
    c = jnp.where(c >= 0, c, 0.2 * c)
    c_ref[...] = jnp.full((128,), c, jnp.float32)


def _combine_kernel(p_ref, b_ref, w_ref, o_ref):
    o = jnp.concatenate([p_ref[0], p_ref[1]], axis=1) + b_ref[...][None, :]
    o_ref[...] = jnp.where(o >= 0, o, w_ref[0, 0] * o)


def _sc_edge_kernel(ev_hbm, asrc_hbm, adst_hbm, c_hbm, xp_hbm,
                    out_hbm,
                    ev, ex, av_s, av_d, dn, rows, cbuf, zbuf, ibuf,
                    sidx, didx, sem,
                    acc, dfinal):
    cid = lax.axis_index("c")
    sid = lax.axis_index("s")
    zero16 = jnp.zeros((16,), jnp.float32)
    rows16 = CH16 // 128

    pltpu.sync_copy(ev_hbm.at[pl.ds(sid * rows16, rows16)], ev)
    pltpu.sync_copy(asrc_hbm, av_s)
    pltpu.sync_copy(adst_hbm, av_d)
    pltpu.sync_copy(c_hbm, cbuf)
    c = cbuf[pl.ds(0, 16)][0]

    def zero_dn(i, carry):
        for q in range(8):
            dn[i, pl.ds(q * 16, 16)] = zero16
        return carry
    lax.fori_loop(0, NPAD // 128, zero_dn, 0)

    for r in range(DROWS // 16):
        for q in range(8):
            zbuf[r, pl.ds(q * 16, 16)] = zero16
    for g in range(DROWS // 16):
        ibuf[pl.ds(g * 16, 16)] = lax.iota(jnp.int32, 16) + g * 16
    pltpu.sync_copy(zbuf, dfinal.at[pl.ds(sid * (DROWS // 16), DROWS // 16)])

    def zero_rows(i, carry):
        r = i // 4
        q = lax.rem(i, 4)
        rows[r, pl.ds(q * 16, 16)] = zero16
        return carry
    lax.fori_loop(0, 128 * 4, zero_rows, 0)
    for k in range(NSLICE // 128):
        pltpu.sync_copy(rows, acc.at[pl.ds(sid * NSLICE + k * 128, 128)])
    plsc.subcore_barrier()

    def edge_body(i, carry):
        for q in range(8):
            w16 = ev[i, pl.ds(q * 16, 16)]
            s16 = w16 & 0xFFFF
            d16 = lax.shift_right_logical(w16, 16)
            a = plsc.load_gather(av_s, [s16]) + plsc.load_gather(av_d, [d16])
            a = jnp.where(a >= 0, a, 0.2 * a) - c
            e16 = jnp.exp(a)
            ex[pl.ds(i * 128 + q * 16, 16)] = e16
            plsc.addupdate_scatter(dn, [d16 >> 7, d16 & 127], e16)
        return carry
    lax.fori_loop(0, rows16, edge_body, 0)

    pltpu.sync_copy(dn, dfinal.at[ibuf], add=True)
    plsc.subcore_barrier()
    pltpu.sync_copy(dfinal, dn)

    def coef_body(k, carry):
        for q in range(8):
            idx = k * 128 + q * 16
            d16 = lax.shift_right_logical(ev[k, pl.ds(q * 16, 16)], 16)
            den = plsc.load_gather(dn, [d16 >> 7, d16 & 127])
            ex[pl.ds(idx, 16)] = ex[pl.ds(idx, 16)] / (den + 1e-16)
        return carry
    lax.fori_loop(0, NCHUNK, coef_body, 0)

    coff = cid * NPAD

    def chunk_body(jc, carry):
        off = jc * 128
        for q in range(8):
            sl = pl.ds(q * 16, 16)
            w16 = ev[jc, sl]
            sidx[sl] = (w16 & 0xFFFF) + coff
            didx[sl] = lax.shift_right_logical(w16, 16)
        pltpu.async_copy(xp_hbm.at[sidx], rows, sem).wait()

        def scale_body(g, c2):
            cgrp = ex[pl.ds(off + g * 16, 16)]
            for u in range(16):
                cv = jnp.full((16,), cgrp[u], jnp.float32)
                for j in range(4):
                    sl = pl.ds(j * 16, 16)
                    rows[g * 16 + u, sl] = rows[g * 16 + u, sl] * cv
            return c2
        lax.fori_loop(0, 8, scale_body, 0)
        pltpu.sync_copy(rows, acc.at[didx], add=True)
        return carry
    lax.fori_loop(0, NCHUNK, chunk_body, 0)

    plsc.subcore_barrier()
    pltpu.sync_copy(acc.at[pl.ds(sid * NSLICE, NSLICE)],
                    out_hbm.at[cid, pl.ds(sid * NSLICE, NSLICE)])


def kernel(x, edge_index, W, att_src, att_dst, bias, prelu_w):
    xpad = jnp.zeros((NPAD, F), jnp.float32).at[:N].set(x)
    loop = jnp.arange(N, dtype=jnp.int32)
    pad = jnp.full((EP_PAD - EP,), N, dtype=jnp.int32)
    srcf = jnp.concatenate([edge_index[0], loop, pad])
    dstf = jnp.concatenate([edge_index[1], loop, pad])
    ev = (srcf | (dstf << 16)).reshape(EP_PAD // 128, 128)

    xp, a_src, a_dst, c = pl.pallas_call(
        _dense_kernel,
        out_shape=(
            jax.ShapeDtypeStruct((2 * NPAD, FH), jnp.float32),
            jax.ShapeDtypeStruct((NPAD,), jnp.float32),
            jax.ShapeDtypeStruct((NPAD,), jnp.float32),
            jax.ShapeDtypeStruct((128,), jnp.float32),
        ),
    )(xpad, W, att_src.reshape(1, F), att_dst.reshape(1, F))

    mesh = plsc.VectorSubcoreMesh(core_axis_name="c", subcore_axis_name="s")
    sc = pl.kernel(
        _sc_edge_kernel,
        out_type=jax.ShapeDtypeStruct((2, NPAD, FH), jnp.float32),
        mesh=mesh,
        compiler_params=pltpu.CompilerParams(needs_layout_passes=False,
                                             use_tc_tiling_on_sc=False),
        scratch_types=[
            pltpu.VMEM((CH16 // 128, 128), jnp.int32),
            pltpu.VMEM((CH16,), jnp.float32),
            pltpu.VMEM((NPAD,), jnp.float32),
            pltpu.VMEM((NPAD,), jnp.float32),
            pltpu.VMEM((DROWS, 128), jnp.float32),
            pltpu.VMEM((128, FH), jnp.float32),
            pltpu.VMEM((128,), jnp.float32),
            pltpu.VMEM((DROWS // 16, 128), jnp.float32),
            pltpu.VMEM((DROWS,), jnp.int32),
            pltpu.VMEM((128,), jnp.int32),
            pltpu.VMEM((128,), jnp.int32),
            pltpu.SemaphoreType.DMA,
            pltpu.VMEM_SHARED((NPAD, FH), jnp.float32),
            pltpu.VMEM_SHARED((DROWS, 128), jnp.float32),
        ],
    )
    partial = sc(ev, a_src, a_dst, c, xp)

    out = pl.pallas_call(
        _combine_kernel,
        out_shape=jax.ShapeDtypeStruct((NPAD, F), jnp.float32),
    )(partial, bias, prelu_w.reshape(1, 1))
    return out[:N]


# speedup vs baseline: 20.4568x; 1.3296x over previous
import jax
import jax.numpy as jnp
from jax import lax
from jax.experimental import pallas as pl
from jax.experimental.pallas import tpu as pltpu
from jax.experimental.pallas import tpu_sc as plsc

N = 10000
F = 128
FH = F // 2
E = 320000
NPAD = 10240
EP = E + N
CH16 = 21504
EP_PAD = 16 * CH16
NCHUNK = CH16 // 128
NSLICE = NPAD // 16
DROWS = NPAD // 128
CHUNK = 64
NCH = CH16 // CHUNK


def _dense_kernel(x_ref, w_ref, as_ref, ad_ref, xp_ref, asrc_ref, adst_ref,
                  c_ref):
    xp = jnp.dot(x_ref[...], w_ref[...], preferred_element_type=jnp.float32)
    xp_ref[:NPAD, :] = xp[:, :FH]
    xp_ref[NPAD:, :] = xp[:, FH:]
    a_s = jnp.sum(xp * as_ref[...], axis=1)
    a_d = jnp.sum(xp * ad_ref[...], axis=1)
    asrc_ref[...] = a_s
    adst_ref[...] = a_d
    c = jnp.max(a_s) + jnp.max(a_d)
    c = jnp.where(c >= 0, c, 0.2 * c)
    c_ref[...] = jnp.full((128,), c, jnp.float32)


def _combine_kernel(p_ref, b_ref, w_ref, o_ref):
    o = jnp.concatenate([p_ref[0], p_ref[1]], axis=1) + b_ref[...][None, :]
    o_ref[...] = jnp.where(o >= 0, o, w_ref[0, 0] * o)


def _sc_edge_kernel(ev_hbm, asrc_hbm, adst_hbm, c_hbm, xp_hbm,
                    out_hbm,
                    ev, ex, av_s, av_d, dn, rows0, rows1, rows2,
                    cbuf, zbuf, ibuf,
                    sidx0, sidx1, sidx2, didx0, didx1, didx2,
                    gsem0, gsem1, gsem2, ssem0, ssem1, ssem2,
                    acc, dfinal):
    cid = lax.axis_index("c")
    sid = lax.axis_index("s")
    zero16 = jnp.zeros((16,), jnp.float32)
    rows16 = CH16 // 128

    pltpu.sync_copy(ev_hbm.at[pl.ds(sid * rows16, rows16)], ev)
    pltpu.sync_copy(asrc_hbm, av_s)
    pltpu.sync_copy(adst_hbm, av_d)
    pltpu.sync_copy(c_hbm, cbuf)
    c = cbuf[pl.ds(0, 16)][0]

    def zero_dn(i, carry):
        for q in range(8):
            dn[i, pl.ds(q * 16, 16)] = zero16
        return carry
    lax.fori_loop(0, NPAD // 128, zero_dn, 0)

    for r in range(DROWS // 16):
        for q in range(8):
            zbuf[r, pl.ds(q * 16, 16)] = zero16
    for g in range(DROWS // 16):
        ibuf[pl.ds(g * 16, 16)] = lax.iota(jnp.int32, 16) + g * 16
    pltpu.sync_copy(zbuf, dfinal.at[pl.ds(sid * (DROWS // 16), DROWS // 16)])

    def zero_rows(i, carry):
        r = i // 4
        q = lax.rem(i, 4)
        rows0[r, pl.ds(q * 16, 16)] = zero16
        return carry
    lax.fori_loop(0, CHUNK * 4, zero_rows, 0)
    for k in range(NSLICE // CHUNK):
        pltpu.sync_copy(rows0, acc.at[pl.ds(sid * NSLICE + k * CHUNK, CHUNK)])
    plsc.subcore_barrier()

    def edge_body(i, carry):
        for q in range(8):
            w16 = ev[i, pl.ds(q * 16, 16)]
            s16 = w16 & 0xFFFF
            d16 = lax.shift_right_logical(w16, 16)
            a = plsc.load_gather(av_s, [s16]) + plsc.load_gather(av_d, [d16])
            a = jnp.where(a >= 0, a, 0.2 * a) - c
            e16 = jnp.exp(a)
            ex[pl.ds(i * 128 + q * 16, 16)] = e16
            plsc.addupdate_scatter(dn, [d16 >> 7, d16 & 127], e16)
        return carry
    lax.fori_loop(0, rows16, edge_body, 0)

    pltpu.sync_copy(dn, dfinal.at[ibuf], add=True)
    plsc.subcore_barrier()
    pltpu.sync_copy(dfinal, dn)

    def coef_body(k, carry):
        for q in range(8):
            idx = k * 128 + q * 16
            d16 = lax.shift_right_logical(ev[k, pl.ds(q * 16, 16)], 16)
            den = plsc.load_gather(dn, [d16 >> 7, d16 & 127])
            ex[pl.ds(idx, 16)] = ex[pl.ds(idx, 16)] / (den + 1e-16)
        return carry
    lax.fori_loop(0, NCHUNK, coef_body, 0)

    coff = cid * NPAD
    ROWS = (rows0, rows1, rows2)
    SIDX = (sidx0, sidx1, sidx2)
    DIDX = (didx0, didx1, didx2)
    GSEM = (gsem0, gsem1, gsem2)
    SSEM = (ssem0, ssem1, ssem2)

    def build_and_gather(jc, r):
        erow = lax.shift_right_logical(jc, 1)
        ecol = (jc & 1) * 64
        for q in range(4):
            sl = pl.ds(q * 16, 16)
            w16 = ev[erow, pl.ds(ecol + q * 16, 16)]
            SIDX[r][sl] = (w16 & 0xFFFF) + coff
            DIDX[r][sl] = lax.shift_right_logical(w16, 16)
        pltpu.async_copy(xp_hbm.at[SIDX[r]], ROWS[r], GSEM[r])

    def wait_sem(r, sem_bank):
        pltpu.make_async_copy(xp_hbm.at[pl.ds(0, CHUNK)], ROWS[r],
                              sem_bank[r]).wait()

    def process(jc, r, wait_prev, prefetch):
        wait_sem(r, GSEM)
        off = jc * CHUNK

        def scale_body(g, c2):
            cgrp = ex[pl.ds(off + g * 16, 16)]
            for u in range(16):
                cv = jnp.full((16,), cgrp[u], jnp.float32)
                for j in range(4):
                    sl = pl.ds(j * 16, 16)
                    ROWS[r][g * 16 + u, sl] = ROWS[r][g * 16 + u, sl] * cv
            return c2
        lax.fori_loop(0, CHUNK // 16, scale_body, 0)
        pltpu.async_copy(ROWS[r], acc.at[DIDX[r]], SSEM[r], add=True)
        r2 = (r + 2) % 3
        if wait_prev:
            wait_sem(r2, SSEM)
        if prefetch:
            build_and_gather(jc + 2, r2)

    build_and_gather(0, 0)
    build_and_gather(1, 1)
    process(0, 0, False, True)

    def pipe_body(i, carry):
        for b in range(3):
            process(1 + 3 * i + b, (1 + b) % 3, True, True)
        return carry
    lax.fori_loop(0, (NCH - 3) // 3, pipe_body, 0)

    process(NCH - 2, (NCH - 2) % 3, True, False)
    process(NCH - 1, (NCH - 1) % 3, True, False)
    wait_sem((NCH - 1) % 3, SSEM)

    plsc.subcore_barrier()
    pltpu.sync_copy(acc.at[pl.ds(sid * NSLICE, NSLICE)],
                    out_hbm.at[cid, pl.ds(sid * NSLICE, NSLICE)])


def kernel(x, edge_index, W, att_src, att_dst, bias, prelu_w):
    xpad = jnp.zeros((NPAD, F), jnp.float32).at[:N].set(x)
    loop = jnp.arange(N, dtype=jnp.int32)
    pad = jnp.full((EP_PAD - EP,), N, dtype=jnp.int32)
    srcf = jnp.concatenate([edge_index[0], loop, pad])
    dstf = jnp.concatenate([edge_index[1], loop, pad])
    ev = (srcf | (dstf << 16)).reshape(EP_PAD // 128, 128)

    xp, a_src, a_dst, c = pl.pallas_call(
        _dense_kernel,
        out_shape=(
            jax.ShapeDtypeStruct((2 * NPAD, FH), jnp.float32),
            jax.ShapeDtypeStruct((NPAD,), jnp.float32),
            jax.ShapeDtypeStruct((NPAD,), jnp.float32),
            jax.ShapeDtypeStruct((128,), jnp.float32),
        ),
    )(xpad, W, att_src.reshape(1, F), att_dst.reshape(1, F))

    mesh = plsc.VectorSubcoreMesh(core_axis_name="c", subcore_axis_name="s")
    sc = pl.kernel(
        _sc_edge_kernel,
        out_type=jax.ShapeDtypeStruct((2, NPAD, FH), jnp.float32),
        mesh=mesh,
        compiler_params=pltpu.CompilerParams(needs_layout_passes=False,
                                             use_tc_tiling_on_sc=False),
        scratch_types=[
            pltpu.VMEM((CH16 // 128, 128), jnp.int32),
            pltpu.VMEM((CH16,), jnp.float32),
            pltpu.VMEM((NPAD,), jnp.float32),
            pltpu.VMEM((NPAD,), jnp.float32),
            pltpu.VMEM((DROWS, 128), jnp.float32),
            pltpu.VMEM((CHUNK, FH), jnp.float32),
            pltpu.VMEM((CHUNK, FH), jnp.float32),
            pltpu.VMEM((CHUNK, FH), jnp.float32),
            pltpu.VMEM((128,), jnp.float32),
            pltpu.VMEM((DROWS // 16, 128), jnp.float32),
            pltpu.VMEM((DROWS,), jnp.int32),
            pltpu.VMEM((CHUNK,), jnp.int32),
            pltpu.VMEM((CHUNK,), jnp.int32),
            pltpu.VMEM((CHUNK,), jnp.int32),
            pltpu.VMEM((CHUNK,), jnp.int32),
            pltpu.VMEM((CHUNK,), jnp.int32),
            pltpu.VMEM((CHUNK,), jnp.int32),
            pltpu.SemaphoreType.DMA,
            pltpu.SemaphoreType.DMA,
            pltpu.SemaphoreType.DMA,
            pltpu.SemaphoreType.DMA,
            pltpu.SemaphoreType.DMA,
            pltpu.SemaphoreType.DMA,
            pltpu.VMEM_SHARED((NPAD, FH), jnp.float32),
            pltpu.VMEM_SHARED((DROWS, 128), jnp.float32),
        ],
    )
    partial = sc(ev, a_src, a_dst, c, xp)

    out = pl.pallas_call(
        _combine_kernel,
        out_shape=jax.ShapeDtypeStruct((NPAD, F), jnp.float32),
    )(partial, bias, prelu_w.reshape(1, 1))
    return out[:N]


# speedup vs baseline: 20.8455x; 1.0190x over previous
import jax
import jax.numpy as jnp
from jax import lax
from jax.experimental import pallas as pl
from jax.experimental.pallas import tpu as pltpu
from jax.experimental.pallas import tpu_sc as plsc

N = 10000
F = 128
FH = F // 2
E = 320000
NPAD = 10240
EP = E + N
CH16 = 21504
EP_PAD = 16 * CH16
NCHUNK = CH16 // 128
NSLICE = NPAD // 16
DROWS = NPAD // 128
CHUNK = 64
NCH = CH16 // CHUNK


def _dense_kernel(x_ref, w_ref, as_ref, ad_ref, xp_ref, asrc_ref, adst_ref,
                  c_ref):
    xp = jnp.dot(x_ref[...], w_ref[...], preferred_element_type=jnp.float32)
    xp_ref[:NPAD, :] = xp[:, :FH]
    xp_ref[NPAD:, :] = xp[:, FH:]
    a_s = jnp.sum(xp * as_ref[...], axis=1)
    a_d = jnp.sum(xp * ad_ref[...], axis=1)
    asrc_ref[...] = a_s
    adst_ref[...] = a_d
    c = jnp.max(a_s) + jnp.max(a_d)
    c = jnp.where(c >= 0, c, 0.2 * c)
    c_ref[...] = jnp.full((128,), c, jnp.float32)


def _combine_kernel(p_ref, b_ref, w_ref, o_ref):
    o = jnp.concatenate([p_ref[0], p_ref[1]], axis=1) + b_ref[...][None, :]
    o_ref[...] = jnp.where(o >= 0, o, w_ref[0, 0] * o)


def _sc_edge_kernel(ev_hbm, asrc_hbm, adst_hbm, c_hbm, xp_hbm,
                    out_hbm,
                    ev, ex, av_s, av_d, dn, rows0, rows1, rows2,
                    cbuf, zbuf, ibuf,
                    sidx0, sidx1, sidx2, didx0, didx1, didx2,
                    invb,
                    gsem0, gsem1, gsem2, ssem0, ssem1, ssem2,
                    acc, dfinal):
    cid = lax.axis_index("c")
    sid = lax.axis_index("s")
    zero16 = jnp.zeros((16,), jnp.float32)
    rows16 = CH16 // 128

    pltpu.sync_copy(ev_hbm.at[pl.ds(sid * rows16, rows16)], ev)
    pltpu.sync_copy(asrc_hbm, av_s)
    pltpu.sync_copy(adst_hbm, av_d)
    pltpu.sync_copy(c_hbm, cbuf)
    c = cbuf[pl.ds(0, 16)][0]

    def zero_dn(i, carry):
        for q in range(8):
            dn[i, pl.ds(q * 16, 16)] = zero16
        return carry
    lax.fori_loop(0, NPAD // 128, zero_dn, 0)

    for r in range(DROWS // 16):
        for q in range(8):
            zbuf[r, pl.ds(q * 16, 16)] = zero16
    for g in range(DROWS // 16):
        ibuf[pl.ds(g * 16, 16)] = lax.iota(jnp.int32, 16) + g * 16
    pltpu.sync_copy(zbuf, dfinal.at[pl.ds(sid * (DROWS // 16), DROWS // 16)])

    def zero_rows(i, carry):
        r = i // 4
        q = lax.rem(i, 4)
        rows0[r, pl.ds(q * 16, 16)] = zero16
        return carry
    lax.fori_loop(0, CHUNK * 4, zero_rows, 0)
    for k in range(NSLICE // CHUNK):
        pltpu.sync_copy(rows0, acc.at[pl.ds(sid * NSLICE + k * CHUNK, CHUNK)])
    plsc.subcore_barrier()

    def edge_body(i, carry):
        for q in range(8):
            w16 = ev[i, pl.ds(q * 16, 16)]
            s16 = w16 & 0xFFFF
            d16 = lax.shift_right_logical(w16, 16)
            a = plsc.load_gather(av_s, [s16]) + plsc.load_gather(av_d, [d16])
            a = jnp.where(a >= 0, a, 0.2 * a) - c
            e16 = jnp.exp(a)
            ex[pl.ds(i * 128 + q * 16, 16)] = e16
            plsc.addupdate_scatter(dn, [d16 >> 7, d16 & 127], e16)
        return carry
    lax.fori_loop(0, rows16, edge_body, 0)

    pltpu.sync_copy(dn, dfinal.at[ibuf], add=True)
    plsc.subcore_barrier()

    coff = cid * NPAD
    ROWS = (rows0, rows1, rows2)
    SIDX = (sidx0, sidx1, sidx2)
    DIDX = (didx0, didx1, didx2)
    GSEM = (gsem0, gsem1, gsem2)
    SSEM = (ssem0, ssem1, ssem2)

    def build_and_gather(jc, r):
        erow = lax.shift_right_logical(jc, 1)
        ecol = (jc & 1) * 64
        for q in range(4):
            sl = pl.ds(q * 16, 16)
            w16 = ev[erow, pl.ds(ecol + q * 16, 16)]
            SIDX[r][sl] = (w16 & 0xFFFF) + coff
            DIDX[r][sl] = lax.shift_right_logical(w16, 16)
        pltpu.async_copy(xp_hbm.at[SIDX[r]], ROWS[r], GSEM[r])

    def wait_sem(r, sem_bank):
        pltpu.make_async_copy(xp_hbm.at[pl.ds(0, CHUNK)], ROWS[r],
                              sem_bank[r]).wait()

    def process(jc, r, wait_prev, prefetch):
        wait_sem(r, GSEM)
        off = jc * CHUNK

        def scale_body(g, c2):
            cgrp = ex[pl.ds(off + g * 16, 16)]
            for u in range(16):
                cv = jnp.full((16,), cgrp[u], jnp.float32)
                for j in range(4):
                    sl = pl.ds(j * 16, 16)
                    ROWS[r][g * 16 + u, sl] = ROWS[r][g * 16 + u, sl] * cv
            return c2
        lax.fori_loop(0, CHUNK // 16, scale_body, 0)
        pltpu.async_copy(ROWS[r], acc.at[DIDX[r]], SSEM[r], add=True)
        r2 = (r + 2) % 3
        if wait_prev:
            wait_sem(r2, SSEM)
        if prefetch:
            build_and_gather(jc + 2, r2)

    build_and_gather(0, 0)
    build_and_gather(1, 1)
    process(0, 0, False, True)

    def pipe_body(i, carry):
        for b in range(3):
            process(1 + 3 * i + b, (1 + b) % 3, True, True)
        return carry
    lax.fori_loop(0, (NCH - 3) // 3, pipe_body, 0)

    process(NCH - 2, (NCH - 2) % 3, True, False)
    process(NCH - 1, (NCH - 1) % 3, True, False)
    wait_sem((NCH - 1) % 3, SSEM)

    plsc.subcore_barrier()
    base = sid * NSLICE
    pltpu.sync_copy(dfinal.at[pl.ds(sid * (DROWS // 16), DROWS // 16)], zbuf)
    for j in range(NSLICE // 16):
        v = zbuf[j // 8, pl.ds((j % 8) * 16, 16)]
        invb[pl.ds(j * 16, 16)] = 1.0 / (v + 1e-16)
    for k in range(NSLICE // CHUNK):
        pltpu.sync_copy(acc.at[pl.ds(base + k * CHUNK, CHUNK)], rows0)
        for g in range(CHUNK // 16):
            ig = invb[pl.ds(k * CHUNK + g * 16, 16)]
            for u in range(16):
                cv = jnp.full((16,), ig[u], jnp.float32)
                for j in range(4):
                    sl = pl.ds(j * 16, 16)
                    rows0[g * 16 + u, sl] = rows0[g * 16 + u, sl] * cv
        pltpu.sync_copy(rows0, out_hbm.at[cid, pl.ds(base + k * CHUNK, CHUNK)])


def kernel(x, edge_index, W, att_src, att_dst, bias, prelu_w):
    xpad = jnp.zeros((NPAD, F), jnp.float32).at[:N].set(x)
    loop = jnp.arange(N, dtype=jnp.int32)
    pad = jnp.full((EP_PAD - EP,), N, dtype=jnp.int32)
    srcf = jnp.concatenate([edge_index[0], loop, pad])
    dstf = jnp.concatenate([edge_index[1], loop, pad])
    ev = (srcf | (dstf << 16)).reshape(EP_PAD // 128, 128)

    xp, a_src, a_dst, c = pl.pallas_call(
        _dense_kernel,
        out_shape=(
            jax.ShapeDtypeStruct((2 * NPAD, FH), jnp.float32),
            jax.ShapeDtypeStruct((NPAD,), jnp.float32),
            jax.ShapeDtypeStruct((NPAD,), jnp.float32),
            jax.ShapeDtypeStruct((128,), jnp.float32),
        ),
    )(xpad, W, att_src.reshape(1, F), att_dst.reshape(1, F))

    mesh = plsc.VectorSubcoreMesh(core_axis_name="c", subcore_axis_name="s")
    sc = pl.kernel(
        _sc_edge_kernel,
        out_type=jax.ShapeDtypeStruct((2, NPAD, FH), jnp.float32),
        mesh=mesh,
        compiler_params=pltpu.CompilerParams(needs_layout_passes=False,
                                             use_tc_tiling_on_sc=False),
        scratch_types=[
            pltpu.VMEM((CH16 // 128, 128), jnp.int32),
            pltpu.VMEM((CH16,), jnp.float32),
            pltpu.VMEM((NPAD,), jnp.float32),
            pltpu.VMEM((NPAD,), jnp.float32),
            pltpu.VMEM((DROWS, 128), jnp.float32),
            pltpu.VMEM((CHUNK, FH), jnp.float32),
            pltpu.VMEM((CHUNK, FH), jnp.float32),
            pltpu.VMEM((CHUNK, FH), jnp.float32),
            pltpu.VMEM((128,), jnp.float32),
            pltpu.VMEM((DROWS // 16, 128), jnp.float32),
            pltpu.VMEM((DROWS,), jnp.int32),
            pltpu.VMEM((CHUNK,), jnp.int32),
            pltpu.VMEM((CHUNK,), jnp.int32),
            pltpu.VMEM((CHUNK,), jnp.int32),
            pltpu.VMEM((CHUNK,), jnp.int32),
            pltpu.VMEM((CHUNK,), jnp.int32),
            pltpu.VMEM((CHUNK,), jnp.int32),
            pltpu.VMEM((NSLICE,), jnp.float32),
            pltpu.SemaphoreType.DMA,
            pltpu.SemaphoreType.DMA,
            pltpu.SemaphoreType.DMA,
            pltpu.SemaphoreType.DMA,
            pltpu.SemaphoreType.DMA,
            pltpu.SemaphoreType.DMA,
            pltpu.VMEM_SHARED((NPAD, FH), jnp.float32),
            pltpu.VMEM_SHARED((DROWS, 128), jnp.float32),
        ],
    )
    partial = sc(ev, a_src, a_dst, c, xp)

    out = pl.pallas_call(
        _combine_kernel,
        out_shape=jax.ShapeDtypeStruct((NPAD, F), jnp.float32),
    )(partial, bias, prelu_w.reshape(1, 1))
    return out[:N]
